# baseline (speedup 1.0000x reference)
import functools

import jax
import jax.numpy as jnp
from jax import lax
from jax.experimental import pallas as pl
from jax.experimental.pallas import tpu as pltpu
from jax.experimental.pallas import tpu_sc as plsc

_NUM_CORES = 2
_NUM_SUBCORES = 16
_NW = _NUM_CORES * _NUM_SUBCORES

_NEG = -1e30


def _make_ell_body(grid, d):
    def _ell_body(ta_ref, tb_ref, w1_ref, b1_ref, wq_ref, out_ref, tabp_ref):
        ident = (lax.broadcasted_iota(jnp.int32, (d, d), 0)
                 == lax.broadcasted_iota(jnp.int32, (d, d), 1)
                 ).astype(jnp.float32)
        for half, (t_ref, row_off) in enumerate(((ta_ref, 0), (tb_ref, grid))):
            x = t_ref[...]
            z = (jax.lax.dot_general(
                    w1_ref[...], x, (((0,), (0,)), ((), ())),
                    preferred_element_type=jnp.float32)
                 + b1_ref[...])
            h = 1.0 / (1.0 + jnp.exp(-z))
            row = jax.lax.dot_general(
                wq_ref[...], h, (((0,), (0,)), ((), ())),
                preferred_element_type=jnp.float32)
            out_ref[pl.ds(pl.program_id(0) + row_off, 1), :] = row
            xt = jnp.transpose(x)
            tabp_ref[:, pl.ds(half * d, d)] = xt
    return _ell_body


def _compute_ell(table, W1, b1, wq):
    n, d = table.shape
    cols = 2048
    grid = pl.cdiv(n, 2 * cols)
    half_n = grid * cols
    last_blk = (n - 1) // cols
    tab_t = table.T
    ell, tabp = pl.pallas_call(
        _make_ell_body(grid, d),
        grid=(grid,),
        in_specs=[
            pl.BlockSpec((d, cols), lambda i: (0, i)),
            pl.BlockSpec((d, cols),
                         lambda i, g=grid, lb=last_blk: (0, jnp.minimum(i + g, lb))),
            pl.BlockSpec((d, d), lambda i: (0, 0)),
            pl.BlockSpec((d, 1), lambda i: (0, 0)),
            pl.BlockSpec((d, 1), lambda i: (0, 0)),
        ],
        out_specs=[
            pl.BlockSpec((2 * grid, cols), lambda i: (0, 0)),
            pl.BlockSpec((cols, 2 * d), lambda i: (i, 0)),
        ],
        out_shape=[
            jax.ShapeDtypeStruct((2 * grid, cols), jnp.float32),
            jax.ShapeDtypeStruct((half_n, 2 * d), jnp.float32),
        ],
    )(tab_t, tab_t, W1, b1.reshape(d, 1), wq)
    return ell, tabp, half_n


def _sc_pool(tabp, ell, idx, lens, nbask, seq, d, half_n):
    bpw = nbask // _NW
    mesh = plsc.VectorSubcoreMesh(
        core_axis_name="c", subcore_axis_name="s",
        num_cores=_NUM_CORES, num_subcores=_NUM_SUBCORES)

    @functools.partial(
        pl.kernel,
        out_type=jax.ShapeDtypeStruct((nbask, d), jnp.float32),
        mesh=mesh,
        compiler_params=pltpu.CompilerParams(
            needs_layout_passes=False, use_tc_tiling_on_sc=True),
        scratch_types=[
            pltpu.VMEM((bpw * seq + 16,), jnp.int32),
            pltpu.VMEM((bpw * seq,), jnp.int32),
            pltpu.VMEM((bpw + 16,), jnp.int32),
            pltpu.VMEM((2, seq), jnp.float32),
            pltpu.VMEM((2, seq, 128), jnp.float32),
            pltpu.VMEM((seq + 16,), jnp.float32),
            pltpu.VMEM((seq + 16,), jnp.int32),
            pltpu.VMEM((bpw, d), jnp.float32),
            pltpu.SemaphoreType.DMA,
            pltpu.SemaphoreType.DMA,
        ],
    )
    def body(tabp_hbm, ell_hbm, idx_hbm, len_hbm, out_hbm,
             idx_v, idxq_v, len_v, log_v, rows_v, alpha_v, off_v, out_v,
             sem0, sem1):
        wid = lax.axis_index("s") * _NUM_CORES + lax.axis_index("c")
        base = wid * bpw
        pltpu.sync_copy(idx_hbm.at[pl.ds(base * seq, bpw * seq)],
                        idx_v.at[pl.ds(0, bpw * seq)])
        pltpu.sync_copy(len_hbm.at[pl.ds(base, bpw)], len_v.at[pl.ds(0, bpw)])

        @pl.loop(0, bpw * seq // 16)
        def _(i):
            v = idx_v[pl.ds(i * 16, 16)]
            ge = (v >= half_n).astype(jnp.int32)
            idxq_v[pl.ds(i * 16, 16)] = v - ge * half_n

        sems = (sem0, sem1)
        rbufs = (rows_v.at[0], rows_v.at[1])
        lbufs = (log_v.at[0], log_v.at[1])

        nbuckets = seq // 16

        def start(b, p):
            nch = (len_v[pl.ds(b, 16)][0] + 15) >> 4
            for c in range(1, nbuckets + 1):
                @pl.when(nch == c)
                def _():
                    pltpu.async_copy(
                        tabp_hbm.at[idxq_v.at[pl.ds(b * seq, c * 16)]],
                        rbufs[p].at[pl.ds(0, c * 16)], sems[p])
            pltpu.async_copy(
                ell_hbm.at[idx_v.at[pl.ds(b * seq, seq)]], lbufs[p], sems[p])

        def wait(b, p):
            nch = (len_v[pl.ds(b, 16)][0] + 15) >> 4
            for c in range(1, nbuckets + 1):
                @pl.when(nch == c)
                def _():
                    pltpu.make_async_copy(
                        tabp_hbm.at[idxq_v.at[pl.ds(b * seq, c * 16)]],
                        rbufs[p].at[pl.ds(0, c * 16)], sems[p]).wait()
            pltpu.make_async_copy(
                ell_hbm.at[idx_v.at[pl.ds(b * seq, seq)]], lbufs[p],
                sems[p]).wait()

        iota = lax.iota(jnp.int32, 16)

        def compute(b, p):
            ln = len_v[pl.ds(b, 16)][0]
            lref = lbufs[p]
            rref = rbufs[p]
            mls = []
            for k in range(4):
                lv = lref[pl.ds(k * 16, 16)]
                msk = (iota + (k * 16)) < ln
                mls.append(jnp.where(msk, lv, _NEG))
            mx = jnp.max(jnp.maximum(jnp.maximum(mls[0], mls[1]),
                                     jnp.maximum(mls[2], mls[3])))
            es = []
            for k in range(4):
                msk = (iota + (k * 16)) < ln
                e = jnp.where(msk, jnp.exp(mls[k] - mx), jnp.float32(0.0))
                es.append(e)
            den = jnp.sum(es[0] + es[1] + es[2] + es[3])
            denv = jnp.full((16,), den, jnp.float32)
            inv = 1.0 / jnp.maximum(denv, jnp.float32(1e-20))
            for k in range(4):
                alpha_v[pl.ds(k * 16, 16)] = es[k] * inv
                iv = idx_v[pl.ds(b * seq + k * 16, 16)]
                off_v[pl.ds(k * 16, 16)] = (iv >= half_n).astype(jnp.int32) * 64
            z = jnp.zeros((16,), jnp.float32)
            nch = (ln + 15) >> 4

            def acc(c, carry):
                a0, a1, a2, a3 = carry
                av = alpha_v[pl.ds(c * 16, 16)]
                hv = off_v[pl.ds(c * 16, 16)]
                for j in range(16):
                    l = c * 16 + j
                    al = av[j]
                    h = hv[j]
                    a0 = a0 + al * rref[l, pl.ds(h, 16)]
                    a1 = a1 + al * rref[l, pl.ds(h + 16, 16)]
                    a2 = a2 + al * rref[l, pl.ds(h + 32, 16)]
                    a3 = a3 + al * rref[l, pl.ds(h + 48, 16)]
                return (a0, a1, a2, a3)

            a0, a1, a2, a3 = lax.fori_loop(0, nch, acc, (z, z, z, z))
            out_v[b, pl.ds(0, 16)] = a0
            out_v[b, pl.ds(16, 16)] = a1
            out_v[b, pl.ds(32, 16)] = a2
            out_v[b, pl.ds(48, 16)] = a3

        start(0, 0)

        @pl.loop(0, bpw // 2)
        def _(i):
            b0 = i * 2
            b1 = b0 + 1
            start(b1, 1)
            wait(b0, 0)
            compute(b0, 0)
            nxt = jnp.minimum(b1 + 1, bpw - 1)
            start(nxt, 0)
            wait(b1, 1)
            compute(b1, 1)

        wait(bpw - 1, 0)
        pltpu.sync_copy(out_v, out_hbm.at[pl.ds(base, bpw)])

    return body(tabp, ell, idx, lens)


def kernel(inputs, length_data, table, W1, b1, wq, bq):
    del bq
    b, nb, seq = inputs.shape
    d = table.shape[1]
    nbask = b * nb
    ell2d, tabp, half_n = _compute_ell(table, W1, b1, wq)
    ell = ell2d.reshape(-1)
    idx = inputs.reshape(nbask * seq).astype(jnp.int32)
    lens = length_data.reshape(nbask).astype(jnp.int32)
    out = _sc_pool(tabp, ell, idx, lens, nbask, seq, d, half_n)
    return out.reshape(b, nb, d)


# speedup vs baseline: 1.8682x; 1.8682x over previous
import functools

import jax
import jax.numpy as jnp
from jax import lax
from jax.experimental import pallas as pl
from jax.experimental.pallas import tpu as pltpu
from jax.experimental.pallas import tpu_sc as plsc

_NUM_CORES = 2
_NUM_SUBCORES = 16
_NW = _NUM_CORES * _NUM_SUBCORES

_NEG = -1e30


def _make_ell_body(grid, d):
    def _ell_body(ta_ref, tb_ref, w1_ref, b1_ref, wq_ref, out_ref, tabp_ref):
        ident = (lax.broadcasted_iota(jnp.int32, (d, d), 0)
                 == lax.broadcasted_iota(jnp.int32, (d, d), 1)
                 ).astype(jnp.float32)
        for half, (t_ref, row_off) in enumerate(((ta_ref, 0), (tb_ref, grid))):
            x = t_ref[...]
            z = (jax.lax.dot_general(
                    w1_ref[...], x, (((0,), (0,)), ((), ())),
                    preferred_element_type=jnp.float32)
                 + b1_ref[...])
            h = 1.0 / (1.0 + jnp.exp(-z))
            row = jax.lax.dot_general(
                wq_ref[...], h, (((0,), (0,)), ((), ())),
                preferred_element_type=jnp.float32)
            out_ref[pl.ds(pl.program_id(0) + row_off, 1), :] = row
            xt = jnp.transpose(x)
            tabp_ref[:, pl.ds(half * d, d)] = xt
    return _ell_body


def _compute_ell(table, W1, b1, wq):
    n, d = table.shape
    cols = 2048
    grid = pl.cdiv(n, 2 * cols)
    half_n = grid * cols
    last_blk = (n - 1) // cols
    tab_t = table.T
    ell, tabp = pl.pallas_call(
        _make_ell_body(grid, d),
        grid=(grid,),
        in_specs=[
            pl.BlockSpec((d, cols), lambda i: (0, i)),
            pl.BlockSpec((d, cols),
                         lambda i, g=grid, lb=last_blk: (0, jnp.minimum(i + g, lb))),
            pl.BlockSpec((d, d), lambda i: (0, 0)),
            pl.BlockSpec((d, 1), lambda i: (0, 0)),
            pl.BlockSpec((d, 1), lambda i: (0, 0)),
        ],
        out_specs=[
            pl.BlockSpec((2 * grid, cols), lambda i: (0, 0)),
            pl.BlockSpec((cols, 2 * d), lambda i: (i, 0)),
        ],
        out_shape=[
            jax.ShapeDtypeStruct((2 * grid, cols), jnp.float32),
            jax.ShapeDtypeStruct((half_n, 2 * d), jnp.float32),
        ],
    )(tab_t, tab_t, W1, b1.reshape(d, 1), wq)
    return ell, tabp, half_n


def _sc_pool(tabp, ell, idx, lens, nbask, seq, d, half_n):
    bpw = nbask // _NW
    mesh = plsc.VectorSubcoreMesh(
        core_axis_name="c", subcore_axis_name="s",
        num_cores=_NUM_CORES, num_subcores=_NUM_SUBCORES)

    @functools.partial(
        pl.kernel,
        out_type=jax.ShapeDtypeStruct((nbask, d), jnp.float32),
        mesh=mesh,
        compiler_params=pltpu.CompilerParams(
            needs_layout_passes=False, use_tc_tiling_on_sc=True),
        scratch_types=[
            pltpu.VMEM((bpw * seq + 16,), jnp.int32),
            pltpu.VMEM((bpw * seq,), jnp.int32),
            pltpu.VMEM((bpw + 16,), jnp.int32),
            pltpu.VMEM((2, seq), jnp.float32),
            pltpu.VMEM((2, seq, 128), jnp.float32),
            pltpu.VMEM((seq + 16,), jnp.float32),
            pltpu.VMEM((seq + 16,), jnp.int32),
            pltpu.VMEM((bpw, d), jnp.float32),
            pltpu.SemaphoreType.DMA,
            pltpu.SemaphoreType.DMA,
        ],
    )
    def body(tabp_hbm, ell_hbm, idx_hbm, len_hbm, out_hbm,
             idx_v, idxq_v, len_v, log_v, rows_v, alpha_v, off_v, out_v,
             sem0, sem1):
        wid = lax.axis_index("s") * _NUM_CORES + lax.axis_index("c")
        base = wid * bpw
        pltpu.sync_copy(idx_hbm.at[pl.ds(base * seq, bpw * seq)],
                        idx_v.at[pl.ds(0, bpw * seq)])
        pltpu.sync_copy(len_hbm.at[pl.ds(base, bpw)], len_v.at[pl.ds(0, bpw)])

        @pl.loop(0, bpw * seq // 16)
        def _(i):
            v = idx_v[pl.ds(i * 16, 16)]
            ge = (v >= half_n).astype(jnp.int32)
            idxq_v[pl.ds(i * 16, 16)] = v - ge * half_n

        sems = (sem0, sem1)
        rbufs = (rows_v.at[0], rows_v.at[1])
        lbufs = (log_v.at[0], log_v.at[1])

        def start(b, p):
            pltpu.async_copy(
                tabp_hbm.at[idxq_v.at[pl.ds(b * seq, seq)]], rbufs[p], sems[p])
            pltpu.async_copy(
                ell_hbm.at[idx_v.at[pl.ds(b * seq, seq)]], lbufs[p], sems[p])

        def wait(b, p):
            pltpu.make_async_copy(
                tabp_hbm.at[idxq_v.at[pl.ds(b * seq, seq)]], rbufs[p],
                sems[p]).wait()
            pltpu.make_async_copy(
                ell_hbm.at[idx_v.at[pl.ds(b * seq, seq)]], lbufs[p],
                sems[p]).wait()

        iota = lax.iota(jnp.int32, 16)

        def compute(b, p):
            ln = len_v[pl.ds(b, 16)][0]
            lref = lbufs[p]
            rref = rbufs[p]
            mls = []
            for k in range(4):
                lv = lref[pl.ds(k * 16, 16)]
                msk = (iota + (k * 16)) < ln
                mls.append(jnp.where(msk, lv, _NEG))
            mx = jnp.max(jnp.maximum(jnp.maximum(mls[0], mls[1]),
                                     jnp.maximum(mls[2], mls[3])))
            es = []
            for k in range(4):
                msk = (iota + (k * 16)) < ln
                e = jnp.where(msk, jnp.exp(mls[k] - mx), jnp.float32(0.0))
                es.append(e)
            den = jnp.sum(es[0] + es[1] + es[2] + es[3])
            denv = jnp.full((16,), den, jnp.float32)
            inv = 1.0 / jnp.maximum(denv, jnp.float32(1e-20))
            for k in range(4):
                alpha_v[pl.ds(k * 16, 16)] = es[k] * inv
                iv = idx_v[pl.ds(b * seq + k * 16, 16)]
                off_v[pl.ds(k * 16, 16)] = (iv >= half_n).astype(jnp.int32) * 64
            z = jnp.zeros((16,), jnp.float32)

            def acc(l, carry):
                a0, a1, a2, a3 = carry
                al = alpha_v[pl.ds(l, 16)][0]
                h = off_v[pl.ds(l, 16)][0]
                a0 = a0 + al * rref[l, pl.ds(h, 16)]
                a1 = a1 + al * rref[l, pl.ds(h + 16, 16)]
                a2 = a2 + al * rref[l, pl.ds(h + 32, 16)]
                a3 = a3 + al * rref[l, pl.ds(h + 48, 16)]
                return (a0, a1, a2, a3)

            a0, a1, a2, a3 = lax.fori_loop(0, ln, acc, (z, z, z, z))
            out_v[b, pl.ds(0, 16)] = a0
            out_v[b, pl.ds(16, 16)] = a1
            out_v[b, pl.ds(32, 16)] = a2
            out_v[b, pl.ds(48, 16)] = a3

        start(0, 0)

        @pl.loop(0, bpw // 2)
        def _(i):
            b0 = i * 2
            b1 = b0 + 1
            start(b1, 1)
            wait(b0, 0)
            compute(b0, 0)
            nxt = jnp.minimum(b1 + 1, bpw - 1)
            start(nxt, 0)
            wait(b1, 1)
            compute(b1, 1)

        wait(bpw - 1, 0)
        pltpu.sync_copy(out_v, out_hbm.at[pl.ds(base, bpw)])

    return body(tabp, ell, idx, lens)


def kernel(inputs, length_data, table, W1, b1, wq, bq):
    del bq
    b, nb, seq = inputs.shape
    d = table.shape[1]
    nbask = b * nb
    ell2d, tabp, half_n = _compute_ell(table, W1, b1, wq)
    ell = ell2d.reshape(-1)
    idx = inputs.reshape(nbask * seq).astype(jnp.int32)
    lens = length_data.reshape(nbask).astype(jnp.int32)
    out = _sc_pool(tabp, ell, idx, lens, nbask, seq, d, half_n)
    return out.reshape(b, nb, d)
